# baseline (speedup 1.0000x reference)
import functools

import jax
import jax.numpy as jnp
from jax import lax
from jax.experimental import pallas as pl
from jax.experimental.pallas import tpu as pltpu
from jax.experimental.pallas import tpu_sc as plsc

B = 16384
DIM = 64
NUM_FEAT = 16
OUT_D = DIM + NUM_FEAT

NC = 2
NS = 16
NW = NC * NS
BPW = B // NW
K = 16
NCHUNKS = BPW // K


@functools.partial(
    pl.kernel,
    out_type=jax.ShapeDtypeStruct((B, OUT_D), jnp.float32),
    mesh=plsc.VectorSubcoreMesh(core_axis_name="c", subcore_axis_name="s"),
    scratch_types=[
        pltpu.VMEM((BPW,), jnp.int32),
        pltpu.VMEM((K, 8, DIM), jnp.float32),
        pltpu.VMEM((BPW * NUM_FEAT,), jnp.float32),
        pltpu.VMEM((BPW, OUT_D), jnp.float32),
        pltpu.SemaphoreType.DMA,
    ],
)
def _sc_gather_concat(idx_hbm, num_hbm, table_hbm, out_hbm,
                      idx_v, slab_v, num_v, out_v, sem):
    wid = lax.axis_index("s") * NC + lax.axis_index("c")
    base = wid * BPW

    pltpu.sync_copy(idx_hbm.at[pl.ds(base, BPW)], idx_v)
    pltpu.sync_copy(num_hbm.at[wid], num_v)

    def chunk(c, carry):
        i0 = c * K
        vidx = idx_v[pl.ds(i0, K)]
        vrow0 = vidx & -8
        vrem = vidx & 7
        copies = []
        for j in range(K):
            row0 = pl.multiple_of(vrow0[j], 8)
            copies.append(pltpu.async_copy(
                table_hbm.at[pl.ds(row0, 8)], slab_v.at[j], sem))
        for j in range(K):
            copies[j].wait()
            rem = vrem[j]
            for k in range(DIM // 16):
                out_v[i0 + j, pl.ds(k * 16, 16)] = slab_v[j, rem, pl.ds(k * 16, 16)]
            out_v[i0 + j, pl.ds(DIM, NUM_FEAT)] = num_v[pl.ds((i0 + j) * NUM_FEAT, NUM_FEAT)]
        return carry

    lax.fori_loop(0, NCHUNKS, chunk, 0)

    pltpu.sync_copy(out_v, out_hbm.at[pl.ds(base, BPW)])


def kernel(c_emb_input, c_numeric, emb_table):
    idx = c_emb_input.astype(jnp.int32)
    num = c_numeric.reshape(NW, BPW * NUM_FEAT)
    return _sc_gather_concat(idx, num, emb_table)


# speedup vs baseline: 1.7570x; 1.7570x over previous
import functools

import jax
import jax.numpy as jnp
from jax import lax
from jax.experimental import pallas as pl
from jax.experimental.pallas import tpu as pltpu
from jax.experimental.pallas import tpu_sc as plsc

B = 16384
N_TAB = 1000001
DIM = 64
NUM_FEAT = 16
OUT_D = DIM + NUM_FEAT

NC = 2
NS = 16
NW = NC * NS
BPW = B // NW
L = 16
K = 8
TAIL_C = 999936 // 128


@functools.partial(
    pl.kernel,
    out_type=jax.ShapeDtypeStruct((OUT_D, B), jnp.float32),
    mesh=plsc.VectorSubcoreMesh(core_axis_name="c", subcore_axis_name="s"),
    compiler_params=pltpu.CompilerParams(needs_layout_passes=False),
    scratch_types=[
        pltpu.VMEM((BPW,), jnp.int32),
        pltpu.VMEM((K, DIM, 128), jnp.float32),
        pltpu.VMEM((OUT_D, BPW), jnp.float32),
        pltpu.SemaphoreType.DMA,
        pltpu.SemaphoreType.DMA,
    ],
)
def _sc_panel_gather(idx_hbm, numt_hbm, tabt_hbm, tailt_hbm, outt_hbm,
                     idx_v, panel_v, out_v, gsem, nsem):
    wid = lax.axis_index("s") * NC + lax.axis_index("c")
    base = wid * BPW

    pltpu.sync_copy(idx_hbm.at[pl.ds(base, BPW)], idx_v)
    ncopy = pltpu.async_copy(
        numt_hbm.at[:, pl.ds(base, BPW)],
        out_v.at[pl.ds(DIM, NUM_FEAT)], nsem)

    iota = lax.iota(jnp.int32, L)

    def block16(t, carry):
        vidx = idx_v[pl.ds(t * L, L)]
        vchunk = lax.shift_right_logical(vidx, 7)
        vq = vidx & 127

        for h in range(2):
            copies = []
            for j in range(K):
                cs = vchunk[h * K + j]

                @pl.when(cs <= TAIL_C - 1)
                def _():
                    off = pl.multiple_of(cs * 128, 128)
                    pltpu.async_copy(
                        tabt_hbm.at[:, pl.ds(off, 128)], panel_v.at[j], gsem)

                @pl.when(cs >= TAIL_C)
                def _():
                    pltpu.async_copy(tailt_hbm, panel_v.at[j], gsem)

                copies.append(pltpu.make_async_copy(
                    tailt_hbm, panel_v.at[j], gsem))
            for j in range(K):
                copies[j].wait()
                q16 = lax.broadcast(vq[h * K + j], (L,))
                i = t * L + h * K + j
                i16 = lax.broadcast(i, (L,))
                for k in range(DIM // L):
                    col = plsc.load_gather(
                        panel_v, [lax.broadcast(j, (L,)), iota + (k * L), q16])
                    plsc.store_scatter(out_v, [iota + (k * L), i16], col)
        return carry

    lax.fori_loop(0, BPW // L, block16, 0)

    ncopy.wait()
    pltpu.sync_copy(out_v, outt_hbm.at[:, pl.ds(base, BPW)])


def kernel(c_emb_input, c_numeric, emb_table):
    idx = c_emb_input.astype(jnp.int32)
    tabt = emb_table.T
    tailt = jnp.pad(tabt[:, TAIL_C * 128:1000000], ((0, 0), (0, 64)))
    outt = _sc_panel_gather(idx, c_numeric.T, tabt, tailt)
    return outt.T


# speedup vs baseline: 2.1398x; 1.2179x over previous
import functools

import jax
import jax.numpy as jnp
from jax import lax
from jax.experimental import pallas as pl
from jax.experimental.pallas import tpu as pltpu
from jax.experimental.pallas import tpu_sc as plsc

B = 16384
N_TAB = 1000001
DIM = 64
NUM_FEAT = 16
OUT_D = DIM + NUM_FEAT

NC = 2
NS = 16
NW = NC * NS
BPW = B // NW
L = 16
K = 8
TAIL_C = 999936 // 128


@functools.partial(
    pl.kernel,
    out_type=jax.ShapeDtypeStruct((OUT_D, B), jnp.float32),
    mesh=plsc.VectorSubcoreMesh(core_axis_name="c", subcore_axis_name="s"),
    compiler_params=pltpu.CompilerParams(needs_layout_passes=False),
    scratch_types=[
        pltpu.VMEM((BPW + L,), jnp.int32),
        pltpu.VMEM((K, DIM, 128), jnp.float32),
        pltpu.VMEM((OUT_D, BPW), jnp.float32),
        pltpu.SemaphoreType.DMA,
        pltpu.SemaphoreType.DMA,
    ],
)
def _sc_panel_gather(idx_hbm, numt_hbm, tabt_hbm, tailt_hbm, outt_hbm,
                     idx_v, panel_v, out_v, gsem, nsem):
    wid = lax.axis_index("s") * NC + lax.axis_index("c")
    base = wid * BPW

    pltpu.sync_copy(idx_hbm.at[pl.ds(base, BPW)], idx_v.at[pl.ds(0, BPW)])
    ncopy = pltpu.async_copy(
        numt_hbm.at[:, pl.ds(base, BPW)],
        out_v.at[pl.ds(DIM, NUM_FEAT)], nsem)

    iota = lax.iota(jnp.int32, L)

    def fire(cs, j):
        @pl.when(cs <= TAIL_C - 1)
        def _():
            off = pl.multiple_of(cs * 128, 128)
            pltpu.async_copy(
                tabt_hbm.at[:, pl.ds(off, 128)], panel_v.at[j], gsem)

        @pl.when(cs >= TAIL_C)
        def _():
            pltpu.async_copy(tailt_hbm, panel_v.at[j], gsem)

    v0 = idx_v[pl.ds(0, L)]
    vc0 = lax.shift_right_logical(v0, 7)
    for j in range(K):
        fire(vc0[j], j)

    def ring(it, carry):
        i0 = it * K
        v = idx_v[pl.ds(i0, L)]
        vc = lax.shift_right_logical(v, 7)
        vq = v & 127
        for j in range(K):
            pltpu.make_async_copy(tailt_hbm, panel_v.at[j], gsem).wait()
            q16 = lax.broadcast(vq[j], (L,))
            i16 = lax.broadcast(i0 + j, (L,))
            for k in range(DIM // L):
                col = plsc.load_gather(
                    panel_v, [lax.broadcast(j, (L,)), iota + (k * L), q16])
                plsc.store_scatter(out_v, [iota + (k * L), i16], col)

            @pl.when(i0 + K + j < BPW)
            def _():
                fire(vc[K + j], j)
        return carry

    lax.fori_loop(0, BPW // K, ring, 0)

    ncopy.wait()
    pltpu.sync_copy(out_v, outt_hbm.at[:, pl.ds(base, BPW)])


def kernel(c_emb_input, c_numeric, emb_table):
    idx = c_emb_input.astype(jnp.int32)
    tabt = emb_table.T
    tailt = jnp.pad(tabt[:, TAIL_C * 128:1000000], ((0, 0), (0, 64)))
    outt = _sc_panel_gather(idx, c_numeric.T, tabt, tailt)
    return outt.T
